# baseline (speedup 1.0000x reference)
import functools

import jax
import jax.numpy as jnp
from jax import lax
from jax.experimental import pallas as pl
from jax.experimental.pallas import tpu as pltpu
from jax.experimental.pallas import tpu_sc as plsc

B, T, V, S = 32, 64, 10000, 512
COV_LOSS_WT = 1.0
BT = B * T

_NC, _NS, _L = 2, 16, 16
_NW = _NC * _NS
_PER_W = BT // _NW


def _sc_gather(fd_flat, tgt_flat):
    mesh = plsc.VectorSubcoreMesh(core_axis_name="c", subcore_axis_name="s")

    @functools.partial(
        pl.kernel,
        mesh=mesh,
        out_type=jax.ShapeDtypeStruct((BT,), jnp.float32),
        scratch_types=[
            pltpu.VMEM((_PER_W,), jnp.int32),
            pltpu.VMEM((_PER_W,), jnp.float32),
            pltpu.SemaphoreType.DMA,
        ],
    )
    def k(fd_hbm, tgt_hbm, out_hbm, idx_v, gold_v, sem):
        wid = lax.axis_index("s") * _NC + lax.axis_index("c")
        base = wid * _PER_W
        pltpu.sync_copy(tgt_hbm.at[pl.ds(base, _PER_W)], idx_v)
        for i in range(_PER_W // _L):
            pos = lax.iota(jnp.int32, _L) + (base + i * _L)
            idx_v[pl.ds(i * _L, _L)] = idx_v[pl.ds(i * _L, _L)] + pos * V
        pltpu.async_copy(fd_hbm.at[idx_v], gold_v, sem).wait()
        pltpu.sync_copy(gold_v, out_hbm.at[pl.ds(base, _PER_W)])

    return k(fd_flat, tgt_flat)


def _tc_body(gold_ref, mask_ref, attn_ref, out_ref, acc_ref):
    b = pl.program_id(0)
    attn = attn_ref[0]
    row = lax.broadcasted_iota(jnp.int32, (T, T), 0)
    col = lax.broadcasted_iota(jnp.int32, (T, T), 1)
    ltri = (col < row).astype(jnp.float32)
    coverage = jnp.dot(ltri, attn, precision=lax.Precision.HIGHEST,
                       preferred_element_type=jnp.float32)
    covloss = jnp.sum(jnp.minimum(attn, coverage), axis=1, keepdims=True)
    mrow = mask_ref[0]
    grow = gold_ref[0]
    s_cov = jnp.dot(mrow, covloss, precision=lax.Precision.HIGHEST,
                    preferred_element_type=jnp.float32)
    s_nll = jnp.sum(-jnp.log(grow) * mrow, axis=1, keepdims=True)
    dl = jnp.sum(mrow, axis=1, keepdims=True)

    @pl.when(b == 0)
    def _():
        acc_ref[...] = jnp.zeros((1, 1), jnp.float32)

    acc_ref[...] += (s_nll + COV_LOSS_WT * s_cov) / dl

    @pl.when(b == B - 1)
    def _():
        out_ref[...] = acc_ref[...] / B


def _tc_reduce(gold3, mask3, attn, interpret=False):
    return pl.pallas_call(
        _tc_body,
        grid=(B,),
        in_specs=[
            pl.BlockSpec((1, 1, T), lambda b: (b, 0, 0)),
            pl.BlockSpec((1, 1, T), lambda b: (b, 0, 0)),
            pl.BlockSpec((1, T, S), lambda b: (b, 0, 0)),
        ],
        out_specs=pl.BlockSpec((1, 1), lambda b: (0, 0)),
        out_shape=jax.ShapeDtypeStruct((1, 1), jnp.float32),
        scratch_shapes=[pltpu.VMEM((1, 1), jnp.float32)],
        interpret=interpret,
    )(gold3, mask3, attn)


def kernel(final_dists, attn_dists, target_batch, dec_padding_mask):
    gold = _sc_gather(final_dists.reshape(-1), target_batch.reshape(-1))
    gold3 = gold.reshape(B, 1, T)
    mask3 = dec_padding_mask.reshape(B, 1, T)
    out = _tc_reduce(gold3, mask3, attn_dists)
    return out.reshape(())


# speedup vs baseline: 2.3263x; 2.3263x over previous
import functools

import jax
import jax.numpy as jnp
from jax import lax
from jax.experimental import pallas as pl
from jax.experimental.pallas import tpu as pltpu
from jax.experimental.pallas import tpu_sc as plsc

B, T, V, S = 32, 64, 10000, 512
COV_LOSS_WT = 1.0
BT = B * T

_NC, _NS, _L = 2, 16, 16
_NW = _NC * _NS
_PER_W = BT // _NW


def _sc_gather_slices(fd2, tgt_flat):
    mesh = plsc.VectorSubcoreMesh(core_axis_name="c", subcore_axis_name="s")

    @functools.partial(
        pl.kernel,
        mesh=mesh,
        out_type=jax.ShapeDtypeStruct((BT * 128,), jnp.float32),
        scratch_types=[
            pltpu.VMEM((_PER_W,), jnp.int32),
            pltpu.VMEM((_PER_W, 128), jnp.float32),
            pltpu.SemaphoreType.DMA,
        ],
    )
    def k(fd_hbm, tgt_hbm, out_hbm, tgt_v, buf_v, sem):
        wid = lax.axis_index("s") * _NC + lax.axis_index("c")
        base = wid * _PER_W
        pltpu.sync_copy(tgt_hbm.at[pl.ds(base, _PER_W)], tgt_v)
        handles = []
        for i in range(_PER_W // _L):
            t_vec = tgt_v[pl.ds(i * _L, _L)]
            for m in range(_L):
                j = i * _L + m
                c128 = pl.multiple_of((t_vec[m] // 128) * 128, 128)
                handles.append(pltpu.async_copy(
                    fd_hbm.at[base + j, pl.ds(c128, 128)],
                    buf_v.at[j], sem))
        for h in handles:
            h.wait()
        wh = []
        for j in range(_PER_W):
            wh.append(pltpu.async_copy(
                buf_v.at[j], out_hbm.at[pl.ds((base + j) * 128, 128)], sem))
        for h in wh:
            h.wait()

    return k(fd2, tgt_flat)


def _tc_body(slc_ref, tgt_ref, mask_ref, attn_ref, out_ref, acc_ref):
    b = pl.program_id(0)
    attn = attn_ref[0]
    row = lax.broadcasted_iota(jnp.int32, (T, T), 0)
    col = lax.broadcasted_iota(jnp.int32, (T, T), 1)
    ltri = (col < row).astype(jnp.float32)
    coverage = jnp.dot(ltri, attn, precision=lax.Precision.HIGHEST,
                       preferred_element_type=jnp.float32)
    covloss = jnp.sum(jnp.minimum(attn, coverage), axis=1, keepdims=True)
    mrow = mask_ref[0]
    lanes = lax.broadcasted_iota(jnp.int32, (T, 128), 1)
    sel = lanes == (tgt_ref[0] & 127)
    picked = jnp.where(sel, slc_ref[0], jnp.zeros((T, 128), jnp.float32))
    gold = jnp.sum(picked, axis=1, keepdims=True)
    nlog = -jnp.log(gold)
    s_nll = jnp.dot(mrow, nlog, precision=lax.Precision.HIGHEST,
                    preferred_element_type=jnp.float32)
    s_cov = jnp.dot(mrow, covloss, precision=lax.Precision.HIGHEST,
                    preferred_element_type=jnp.float32)
    dl = jnp.sum(mrow, axis=1, keepdims=True)

    @pl.when(b == 0)
    def _():
        acc_ref[...] = jnp.zeros((1, 1), jnp.float32)

    acc_ref[...] += (s_nll + COV_LOSS_WT * s_cov) / dl

    @pl.when(b == B - 1)
    def _():
        out_ref[...] = acc_ref[...] / B


def _tc_reduce(slc3, tgt3, mask3, attn, interpret=False):
    return pl.pallas_call(
        _tc_body,
        grid=(B,),
        in_specs=[
            pl.BlockSpec((1, T, 128), lambda b: (b, 0, 0)),
            pl.BlockSpec((1, T, 1), lambda b: (b, 0, 0)),
            pl.BlockSpec((1, 1, T), lambda b: (b, 0, 0)),
            pl.BlockSpec((1, T, S), lambda b: (b, 0, 0)),
        ],
        out_specs=pl.BlockSpec((1, 1), lambda b: (0, 0)),
        out_shape=jax.ShapeDtypeStruct((1, 1), jnp.float32),
        scratch_shapes=[pltpu.VMEM((1, 1), jnp.float32)],
        interpret=interpret,
    )(slc3, tgt3, mask3, attn)


def kernel(final_dists, attn_dists, target_batch, dec_padding_mask):
    slices = _sc_gather_slices(final_dists.reshape(BT, V),
                               target_batch.reshape(-1))
    slc3 = slices.reshape(B, T, 128)
    tgt3 = target_batch.reshape(B, T, 1)
    mask3 = dec_padding_mask.reshape(B, 1, T)
    out = _tc_reduce(slc3, tgt3, mask3, attn_dists)
    return out.reshape(())


# speedup vs baseline: 2.5289x; 1.0871x over previous
import functools

import jax
import jax.numpy as jnp
from jax import lax
from jax.experimental import pallas as pl
from jax.experimental.pallas import tpu as pltpu
from jax.experimental.pallas import tpu_sc as plsc

B, T, V, S = 32, 64, 10000, 512
COV_LOSS_WT = 1.0
BT = B * T

_NC, _NS, _L = 2, 16, 16
_NW = _NC * _NS
_PER_W = BT // _NW


def _sc_gather_slices(fd2, tgt_flat):
    mesh = plsc.VectorSubcoreMesh(core_axis_name="c", subcore_axis_name="s")

    @functools.partial(
        pl.kernel,
        mesh=mesh,
        out_type=jax.ShapeDtypeStruct((BT * 128,), jnp.float32),
        scratch_types=[
            pltpu.VMEM((_PER_W,), jnp.int32),
            pltpu.VMEM((_PER_W, 128), jnp.float32),
            pltpu.SemaphoreType.DMA,
        ],
    )
    def k(fd_hbm, tgt_hbm, out_hbm, tgt_v, buf_v, sem):
        wid = lax.axis_index("s") * _NC + lax.axis_index("c")
        base = wid * _PER_W
        pltpu.sync_copy(tgt_hbm.at[pl.ds(base, _PER_W)], tgt_v)
        handles = []
        for i in range(_PER_W // _L):
            t_vec = tgt_v[pl.ds(i * _L, _L)]
            for m in range(_L):
                j = i * _L + m
                c128 = pl.multiple_of((t_vec[m] // 128) * 128, 128)
                handles.append(pltpu.async_copy(
                    fd_hbm.at[base + j, pl.ds(c128, 128)],
                    buf_v.at[j], sem))
        for h in handles:
            h.wait()
        wh = []
        for j in range(_PER_W):
            wh.append(pltpu.async_copy(
                buf_v.at[j], out_hbm.at[pl.ds((base + j) * 128, 128)], sem))
        for h in wh:
            h.wait()

    return k(fd2, tgt_flat)


def _tc_cov_body(mask_ref, attn_ref, out_ref, acc_ref):
    b = pl.program_id(0)
    attn = attn_ref[0]
    row = lax.broadcasted_iota(jnp.int32, (T, T), 0)
    col = lax.broadcasted_iota(jnp.int32, (T, T), 1)
    ltri = (col < row).astype(jnp.float32)
    coverage = jnp.dot(ltri, attn, precision=lax.Precision.HIGHEST,
                       preferred_element_type=jnp.float32)
    covloss = jnp.sum(jnp.minimum(attn, coverage), axis=1, keepdims=True)
    mrow = mask_ref[0]
    s_cov = jnp.dot(mrow, covloss, precision=lax.Precision.HIGHEST,
                    preferred_element_type=jnp.float32)
    dl = jnp.sum(mrow, axis=1, keepdims=True)

    @pl.when(b == 0)
    def _():
        acc_ref[...] = jnp.zeros((1, 1), jnp.float32)

    acc_ref[...] += s_cov / dl

    @pl.when(b == B - 1)
    def _():
        out_ref[...] = acc_ref[...]


def _tc_cov(mask3, attn, interpret=False):
    return pl.pallas_call(
        _tc_cov_body,
        grid=(B,),
        in_specs=[
            pl.BlockSpec((1, 1, T), lambda b: (b, 0, 0)),
            pl.BlockSpec((1, T, S), lambda b: (b, 0, 0)),
        ],
        out_specs=pl.BlockSpec((1, 1), lambda b: (0, 0)),
        out_shape=jax.ShapeDtypeStruct((1, 1), jnp.float32),
        scratch_shapes=[pltpu.VMEM((1, 1), jnp.float32)],
        interpret=interpret,
    )(mask3, attn)


def _tc_final_body(slc_ref, tgt_ref, mask_ref, cov_ref, out_ref):
    lanes = lax.broadcasted_iota(jnp.int32, (B, T, 128), 2)
    sel = lanes == (tgt_ref[...] & 127)
    picked = jnp.where(sel, slc_ref[...],
                       jnp.zeros((B, T, 128), jnp.float32))
    gold = jnp.sum(picked, axis=2, keepdims=True)
    mask = mask_ref[...]
    nll = -jnp.log(gold) * mask
    s_nll = jnp.sum(nll, axis=1, keepdims=True)
    dl = jnp.sum(mask, axis=1, keepdims=True)
    per_ex = (s_nll / dl)[:, 0, :]
    tot = jnp.sum(per_ex, axis=0, keepdims=True)
    out_ref[...] = (tot + COV_LOSS_WT * cov_ref[...]) / B


def _tc_final(slc3, tgt3, mask4, cov, interpret=False):
    return pl.pallas_call(
        _tc_final_body,
        out_shape=jax.ShapeDtypeStruct((1, 1), jnp.float32),
        interpret=interpret,
    )(slc3, tgt3, mask4, cov)


def kernel(final_dists, attn_dists, target_batch, dec_padding_mask):
    slices = _sc_gather_slices(final_dists.reshape(BT, V),
                               target_batch.reshape(-1))
    slc3 = slices.reshape(B, T, 128)
    tgt3 = target_batch.reshape(B, T, 1)
    mask3 = dec_padding_mask.reshape(B, 1, T)
    mask4 = dec_padding_mask.reshape(B, T, 1)
    cov = _tc_cov(mask3, attn_dists)
    out = _tc_final(slc3, tgt3, mask4, cov)
    return out.reshape(())


# speedup vs baseline: 3.2048x; 1.2673x over previous
import functools

import jax
import jax.numpy as jnp
from jax import lax
from jax.experimental import pallas as pl
from jax.experimental.pallas import tpu as pltpu
from jax.experimental.pallas import tpu_sc as plsc

B, T, V, S = 32, 64, 10000, 512
COV_LOSS_WT = 1.0
BT = B * T

_NC, _NS, _L = 2, 16, 16
_NW = _NC * _NS
_PER_W = BT // _NW


def _sc_gather_slices(fd2, tgt_flat):
    mesh = plsc.VectorSubcoreMesh(core_axis_name="c", subcore_axis_name="s")

    @functools.partial(
        pl.kernel,
        mesh=mesh,
        out_type=jax.ShapeDtypeStruct((BT * 128,), jnp.float32),
        scratch_types=[
            pltpu.VMEM((_PER_W,), jnp.int32),
            pltpu.VMEM((_PER_W, 128), jnp.float32),
            pltpu.SemaphoreType.DMA,
        ],
    )
    def k(fd_hbm, tgt_hbm, out_hbm, tgt_v, buf_v, sem):
        wid = lax.axis_index("s") * _NC + lax.axis_index("c")
        base = wid * _PER_W
        pltpu.sync_copy(tgt_hbm.at[pl.ds(base, _PER_W)], tgt_v)
        handles = []
        for i in range(_PER_W // _L):
            t_vec = tgt_v[pl.ds(i * _L, _L)]
            for m in range(_L):
                j = i * _L + m
                c128 = pl.multiple_of((t_vec[m] // 128) * 128, 128)
                handles.append(pltpu.async_copy(
                    fd_hbm.at[base + j, pl.ds(c128, 128)],
                    buf_v.at[j], sem))
        for h in handles:
            h.wait()
        wh = []
        for j in range(_PER_W):
            wh.append(pltpu.async_copy(
                buf_v.at[j], out_hbm.at[pl.ds((base + j) * 128, 128)], sem))
        for h in wh:
            h.wait()

    return k(fd2, tgt_flat)


_CB = 4


def _tc_cov_body(mask_ref, attn_ref, out_ref, acc_ref):
    c = pl.program_id(0)
    row = lax.broadcasted_iota(jnp.int32, (T, T), 0)
    col = lax.broadcasted_iota(jnp.int32, (T, T), 1)
    ltri = (col < row).astype(jnp.float32)

    @pl.when(c == 0)
    def _():
        acc_ref[...] = jnp.zeros((1, 1), jnp.float32)

    for bb in range(_CB):
        attn = attn_ref[bb]
        coverage = jnp.dot(ltri, attn, precision=lax.Precision.HIGHEST,
                           preferred_element_type=jnp.float32)
        covloss = jnp.sum(jnp.minimum(attn, coverage), axis=1,
                          keepdims=True)
        mrow = mask_ref[bb]
        s_cov = jnp.dot(mrow, covloss, precision=lax.Precision.HIGHEST,
                        preferred_element_type=jnp.float32)
        dl = jnp.sum(mrow, axis=1, keepdims=True)
        acc_ref[...] += s_cov / dl

    @pl.when(c == B // _CB - 1)
    def _():
        out_ref[...] = acc_ref[...]


def _tc_cov(mask3, attn, interpret=False):
    return pl.pallas_call(
        _tc_cov_body,
        grid=(B // _CB,),
        in_specs=[
            pl.BlockSpec((_CB, 1, T), lambda c: (c, 0, 0)),
            pl.BlockSpec((_CB, T, S), lambda c: (c, 0, 0)),
        ],
        out_specs=pl.BlockSpec((1, 1), lambda c: (0, 0)),
        out_shape=jax.ShapeDtypeStruct((1, 1), jnp.float32),
        scratch_shapes=[pltpu.VMEM((1, 1), jnp.float32)],
        interpret=interpret,
    )(mask3, attn)


def _tc_final_body(slc_ref, tgt_ref, mask_ref, cov_ref, out_ref):
    lanes = lax.broadcasted_iota(jnp.int32, (B, T, 128), 2)
    sel = lanes == (tgt_ref[...] & 127)
    picked = jnp.where(sel, slc_ref[...],
                       jnp.zeros((B, T, 128), jnp.float32))
    gold = jnp.sum(picked, axis=2, keepdims=True)
    mask = mask_ref[...]
    nll = -jnp.log(gold) * mask
    s_nll = jnp.sum(nll, axis=1, keepdims=True)
    dl = jnp.sum(mask, axis=1, keepdims=True)
    per_ex = (s_nll / dl)[:, 0, :]
    tot = jnp.sum(per_ex, axis=0, keepdims=True)
    out_ref[...] = (tot + COV_LOSS_WT * cov_ref[...]) / B


def _tc_final(slc3, tgt3, mask4, cov, interpret=False):
    return pl.pallas_call(
        _tc_final_body,
        out_shape=jax.ShapeDtypeStruct((1, 1), jnp.float32),
        interpret=interpret,
    )(slc3, tgt3, mask4, cov)


def kernel(final_dists, attn_dists, target_batch, dec_padding_mask):
    mask3 = dec_padding_mask.reshape(B, 1, T)
    cov = _tc_cov(mask3, attn_dists)
    slices = _sc_gather_slices(final_dists.reshape(BT, V),
                               target_batch.reshape(-1))
    slc3 = slices.reshape(B, T, 128)
    tgt3 = target_batch.reshape(B, T, 1)
    mask4 = dec_padding_mask.reshape(B, T, 1)
    out = _tc_final(slc3, tgt3, mask4, cov)
    return out.reshape(())


# speedup vs baseline: 4.2540x; 1.3274x over previous
import jax
import jax.numpy as jnp
from jax import lax
from jax.experimental import pallas as pl
from jax.experimental.pallas import tpu as pltpu

B, T, V, S = 32, 64, 10000, 512
COV_LOSS_WT = 1.0
BT = B * T

_CB = 32
_ROWS = _CB * T


def _fused_body(tgt_s, fd_any, tgtv_ref, mask3_ref, mask4_ref, attn_ref,
                out_ref, buf_ref, acc_ref, sem):
    c = pl.program_id(0)

    @pl.when(c == 0)
    def _():
        acc_ref[...] = jnp.zeros((1, 1), jnp.float32)

    row = lax.broadcasted_iota(jnp.int32, (T, T), 0)
    col = lax.broadcasted_iota(jnp.int32, (T, T), 1)
    ltri = (col < row).astype(jnp.float32)
    handles = []
    for bb in range(_CB):
        for m in range(T):
            jj = bb * T + m
            p = c * _ROWS + jj
            tt = tgt_s[p]
            c128 = pl.multiple_of((tt // 128) * 128, 128)
            h = pltpu.make_async_copy(
                fd_any.at[p, pl.ds(c128, 128)], buf_ref.at[jj], sem)
            h.start()
            handles.append(h)
        attn = attn_ref[bb]
        coverage = jnp.dot(ltri, attn, precision=lax.Precision.HIGHEST,
                           preferred_element_type=jnp.float32)
        covloss = jnp.sum(jnp.minimum(attn, coverage), axis=1,
                          keepdims=True)
        mrow = mask3_ref[bb]
        s_cov = jnp.dot(mrow, covloss, precision=lax.Precision.HIGHEST,
                        preferred_element_type=jnp.float32)
        dl = jnp.sum(mrow, axis=1, keepdims=True)
        acc_ref[...] += COV_LOSS_WT * s_cov / dl

    for h in handles:
        h.wait()

    lanes = lax.broadcasted_iota(jnp.int32, (_ROWS, 128), 1)
    sel = lanes == (tgtv_ref[...] & 127)
    picked = jnp.where(sel, buf_ref[...],
                       jnp.zeros((_ROWS, 128), jnp.float32))
    gold = jnp.sum(picked, axis=1, keepdims=True)
    mask4 = mask4_ref[...]
    nll = -jnp.log(gold) * mask4
    nll3 = nll.reshape(_CB, T, 1)
    m3 = mask4.reshape(_CB, T, 1)
    s_nll = jnp.sum(nll3, axis=1, keepdims=True)
    dl2 = jnp.sum(m3, axis=1, keepdims=True)
    per_ex = (s_nll / dl2)[:, 0, :]
    acc_ref[...] += jnp.sum(per_ex, axis=0, keepdims=True)

    @pl.when(c == B // _CB - 1)
    def _():
        out_ref[...] = acc_ref[...] / B


def _fused(tgt_flat, fd2, tgtv, mask3, mask4, attn, interpret=False):
    return pl.pallas_call(
        _fused_body,
        grid=(B // _CB,),
        in_specs=[
            pl.BlockSpec(memory_space=pltpu.SMEM),
            pl.BlockSpec(memory_space=pltpu.MemorySpace.HBM),
            pl.BlockSpec((_ROWS, 1), lambda c: (c, 0)),
            pl.BlockSpec((_CB, 1, T), lambda c: (c, 0, 0)),
            pl.BlockSpec((_ROWS, 1), lambda c: (c, 0)),
            pl.BlockSpec((_CB, T, S), lambda c: (c, 0, 0)),
        ],
        out_specs=pl.BlockSpec((1, 1), lambda c: (0, 0)),
        out_shape=jax.ShapeDtypeStruct((1, 1), jnp.float32),
        scratch_shapes=[
            pltpu.VMEM((_ROWS, 128), jnp.float32),
            pltpu.VMEM((1, 1), jnp.float32),
            pltpu.SemaphoreType.DMA,
        ],
        interpret=interpret,
    )(tgt_flat, fd2, tgtv, mask3, mask4, attn)


def kernel(final_dists, attn_dists, target_batch, dec_padding_mask):
    tgt_flat = target_batch.reshape(-1)
    out = _fused(tgt_flat,
                 final_dists.reshape(BT, V),
                 target_batch.reshape(BT, 1),
                 dec_padding_mask.reshape(B, 1, T),
                 dec_padding_mask.reshape(BT, 1),
                 attn_dists)
    return out.reshape(())


# speedup vs baseline: 4.2580x; 1.0010x over previous
import jax
import jax.numpy as jnp
from jax import lax
from jax.experimental import pallas as pl
from jax.experimental.pallas import tpu as pltpu

B, T, V, S = 32, 64, 10000, 512
COV_LOSS_WT = 1.0
BT = B * T

_CB = 32
_ROWS = _CB * T


def _fused_body(tgt_s, fd_any, tgtv_ref, mask3_ref, mask4_ref, attn_ref,
                out_ref, buf_ref, acc_ref, sem):
    c = pl.program_id(0)

    @pl.when(c == 0)
    def _():
        acc_ref[...] = jnp.zeros((1, 1), jnp.float32)

    row = lax.broadcasted_iota(jnp.int32, (T, T), 0)
    col = lax.broadcasted_iota(jnp.int32, (T, T), 1)
    ltri = (col < row).astype(jnp.float32)
    for bb in range(_CB):
        for m in range(T):
            jj = bb * T + m
            p = c * _ROWS + jj
            tt = tgt_s[p]
            c128 = pl.multiple_of((tt // 128) * 128, 128)
            pltpu.make_async_copy(
                fd_any.at[p, pl.ds(c128, 128)], buf_ref.at[jj], sem).start()
        attn = attn_ref[bb]
        coverage = jnp.dot(ltri, attn, precision=lax.Precision.HIGHEST,
                           preferred_element_type=jnp.float32)
        covloss = jnp.sum(jnp.minimum(attn, coverage), axis=1,
                          keepdims=True)
        mrow = mask3_ref[bb]
        s_cov = jnp.dot(mrow, covloss, precision=lax.Precision.HIGHEST,
                        preferred_element_type=jnp.float32)
        dl = jnp.sum(mrow, axis=1, keepdims=True)
        acc_ref[...] += COV_LOSS_WT * s_cov / dl

    pltpu.make_async_copy(
        fd_any.at[pl.ds(0, _ROWS), pl.ds(0, 128)], buf_ref, sem).wait()

    lanes = lax.broadcasted_iota(jnp.int32, (_ROWS, 128), 1)
    sel = lanes == (tgtv_ref[...] & 127)
    picked = jnp.where(sel, buf_ref[...],
                       jnp.zeros((_ROWS, 128), jnp.float32))
    gold = jnp.sum(picked, axis=1, keepdims=True)
    mask4 = mask4_ref[...]
    nll = -jnp.log(gold) * mask4
    nll3 = nll.reshape(_CB, T, 1)
    m3 = mask4.reshape(_CB, T, 1)
    s_nll = jnp.sum(nll3, axis=1, keepdims=True)
    dl2 = jnp.sum(m3, axis=1, keepdims=True)
    per_ex = (s_nll / dl2)[:, 0, :]
    acc_ref[...] += jnp.sum(per_ex, axis=0, keepdims=True)

    @pl.when(c == B // _CB - 1)
    def _():
        out_ref[...] = acc_ref[...] / B


def _fused(tgt_flat, fd2, tgtv, mask3, mask4, attn, interpret=False):
    return pl.pallas_call(
        _fused_body,
        grid=(B // _CB,),
        in_specs=[
            pl.BlockSpec(memory_space=pltpu.SMEM),
            pl.BlockSpec(memory_space=pltpu.MemorySpace.HBM),
            pl.BlockSpec((_ROWS, 1), lambda c: (c, 0)),
            pl.BlockSpec((_CB, 1, T), lambda c: (c, 0, 0)),
            pl.BlockSpec((_ROWS, 1), lambda c: (c, 0)),
            pl.BlockSpec((_CB, T, S), lambda c: (c, 0, 0)),
        ],
        out_specs=pl.BlockSpec((1, 1), lambda c: (0, 0)),
        out_shape=jax.ShapeDtypeStruct((1, 1), jnp.float32),
        scratch_shapes=[
            pltpu.VMEM((_ROWS, 128), jnp.float32),
            pltpu.VMEM((1, 1), jnp.float32),
            pltpu.SemaphoreType.DMA,
        ],
        interpret=interpret,
    )(tgt_flat, fd2, tgtv, mask3, mask4, attn)


def kernel(final_dists, attn_dists, target_batch, dec_padding_mask):
    tgt_flat = target_batch.reshape(-1)
    out = _fused(tgt_flat,
                 final_dists.reshape(BT, V),
                 target_batch.reshape(BT, 1),
                 dec_padding_mask.reshape(B, 1, T),
                 dec_padding_mask.reshape(BT, 1),
                 attn_dists)
    return out.reshape(())


# speedup vs baseline: 4.8076x; 1.1291x over previous
import jax
import jax.numpy as jnp
from jax import lax
from jax.experimental import pallas as pl
from jax.experimental.pallas import tpu as pltpu

B, T, V, S = 32, 64, 10000, 512
COV_LOSS_WT = 1.0
BT = B * T

_CB = 32
_ROWS = _CB * T


def _fused_body(tgt_s, fd_any, tgtv_ref, mask3_ref, attn_ref,
                out_ref, buf_ref, acc_ref, sem):
    c = pl.program_id(0)

    @pl.when(c == 0)
    def _():
        acc_ref[...] = jnp.zeros((1, 1), jnp.float32)

    row = lax.broadcasted_iota(jnp.int32, (T, T), 0)
    col = lax.broadcasted_iota(jnp.int32, (T, T), 1)
    ltri = (col < row).astype(jnp.float32)
    for bb in range(_CB):
        for m in range(T):
            jj = bb * T + m
            p = c * _ROWS + jj
            tt = tgt_s[p]
            c128 = pl.multiple_of((tt // 128) * 128, 128)
            pltpu.make_async_copy(
                fd_any.at[p, pl.ds(c128, 128)], buf_ref.at[jj], sem).start()
        attn = attn_ref[bb]
        coverage = jnp.dot(ltri, attn,
                           preferred_element_type=jnp.float32)
        covloss = jnp.sum(jnp.minimum(attn, coverage), axis=1,
                          keepdims=True)
        mrow = mask3_ref[bb]
        s_cov = jnp.dot(mrow, covloss,
                        preferred_element_type=jnp.float32)
        dl = jnp.sum(mrow, axis=1, keepdims=True)
        acc_ref[...] += COV_LOSS_WT * s_cov / dl

    pltpu.make_async_copy(
        fd_any.at[pl.ds(0, _ROWS), pl.ds(0, 128)], buf_ref, sem).wait()

    lanes = lax.broadcasted_iota(jnp.int32, (_ROWS, 128), 1)
    sel = lanes == (tgtv_ref[...] & 127)
    picked = jnp.where(sel, buf_ref[...],
                       jnp.zeros((_ROWS, 128), jnp.float32))
    gold = jnp.sum(picked, axis=1, keepdims=True)
    nlog = -jnp.log(gold)
    for bb in range(_CB):
        ncol = nlog[bb * T:(bb + 1) * T]
        mrow = mask3_ref[bb]
        s_nll = jnp.dot(mrow, ncol,
                        preferred_element_type=jnp.float32)
        dl = jnp.sum(mrow, axis=1, keepdims=True)
        acc_ref[...] += s_nll / dl

    @pl.when(c == B // _CB - 1)
    def _():
        out_ref[...] = acc_ref[...] / B


def _fused(tgt_flat, fd2, tgtv, mask3, attn, interpret=False):
    return pl.pallas_call(
        _fused_body,
        grid=(B // _CB,),
        in_specs=[
            pl.BlockSpec(memory_space=pltpu.SMEM),
            pl.BlockSpec(memory_space=pltpu.MemorySpace.HBM),
            pl.BlockSpec((_ROWS, 1), lambda c: (c, 0)),
            pl.BlockSpec((_CB, 1, T), lambda c: (c, 0, 0)),
            pl.BlockSpec((_CB, T, S), lambda c: (c, 0, 0)),
        ],
        out_specs=pl.BlockSpec((1, 1), lambda c: (0, 0)),
        out_shape=jax.ShapeDtypeStruct((1, 1), jnp.float32),
        scratch_shapes=[
            pltpu.VMEM((_ROWS, 128), jnp.float32),
            pltpu.VMEM((1, 1), jnp.float32),
            pltpu.SemaphoreType.DMA,
        ],
        interpret=interpret,
    )(tgt_flat, fd2, tgtv, mask3, attn)


def kernel(final_dists, attn_dists, target_batch, dec_padding_mask):
    tgt_flat = target_batch.reshape(-1)
    out = _fused(tgt_flat,
                 final_dists.reshape(BT, V),
                 target_batch.reshape(BT, 1),
                 dec_padding_mask.reshape(B, 1, T),
                 attn_dists)
    return out.reshape(())
